# baseline (speedup 1.0000x reference)
import functools

import numpy as np

import jax
import jax.numpy as jnp
from jax import lax
from jax.experimental import pallas as pl
from jax.experimental.pallas import tpu as pltpu
from jax.experimental.pallas import tpu_sc as plsc

_N = 10000
_E = 320000
_D = 128
_H = 16
_C = 10

_NW = 32
_CH = 1024
_RB = _CH // 128
_PW = 10240
_EP = _NW * _PW
_NCH = _PW // _CH
_NP = 10240
_NPS = _NP // 16

_mesh = plsc.VectorSubcoreMesh(core_axis_name="c", subcore_axis_name="s")


@functools.partial(
    pl.kernel,
    out_type=jax.ShapeDtypeStruct((2 * _NP,), jnp.float32),
    scratch_types=[
        pltpu.VMEM((_RB, 128), jnp.int32),
        pltpu.VMEM((_CH,), jnp.float32),
        pltpu.VMEM((_NPS,), jnp.float32),
        pltpu.VMEM_SHARED((_NP,), jnp.float32),
    ],
    mesh=_mesh,
    compiler_params=pltpu.CompilerParams(use_tc_tiling_on_sc=False),
)
def _deg_kernel(dst_hbm, w_hbm, out_hbm, dst_ref, w_ref, zb, deg_sh):
    c = lax.axis_index("c")
    s = lax.axis_index("s")
    wid = c * 16 + s

    def zrow(i, carry):
        zb[pl.ds(i * 16, 16)] = jnp.zeros((16,), jnp.float32)
        return carry

    lax.fori_loop(0, _NPS // 16, zrow, 0)
    pltpu.sync_copy(zb, deg_sh.at[pl.ds(s * _NPS, _NPS)])
    plsc.subcore_barrier()

    def chunk(ci, carry):
        e0 = wid * _PW + ci * _CH
        r0 = wid * (_PW // 128) + ci * _RB
        pltpu.sync_copy(dst_hbm.at[pl.ds(r0, _RB)], dst_ref)
        pltpu.sync_copy(w_hbm.at[pl.ds(e0, _CH)], w_ref)
        for j in range(_RB):
            pltpu.sync_copy(
                w_ref.at[pl.ds(j * 128, 128)],
                deg_sh.at[dst_ref.at[j]],
                add=True,
            )
        return carry

    lax.fori_loop(0, _NCH, chunk, 0)
    plsc.subcore_barrier()
    pltpu.sync_copy(
        deg_sh.at[pl.ds(s * _NPS, _NPS)],
        out_hbm.at[pl.ds(c * _NP + s * _NPS, _NPS)],
    )


_NRS = _N // 16


@functools.partial(
    pl.kernel,
    out_type=[
        jax.ShapeDtypeStruct((_N, _H), jnp.float32),
        jax.ShapeDtypeStruct((_N, _H), jnp.float32),
    ],
    scratch_types=[
        pltpu.VMEM((2, _RB, 128), jnp.int32),
        pltpu.VMEM((2, _RB, 128), jnp.int32),
        pltpu.VMEM((2, _CH), jnp.float32),
        pltpu.VMEM((2, _CH, _H), jnp.float32),
        pltpu.VMEM_SHARED((_N, _H), jnp.float32),
        pltpu.SemaphoreType.DMA((2,)),
        pltpu.SemaphoreType.DMA((2,)),
    ],
    mesh=_mesh,
    compiler_params=pltpu.CompilerParams(
        use_tc_tiling_on_sc=False, needs_layout_passes=False),
)
def _agg_kernel(tab_hbm, src_hbm, dst_hbm, w_hbm, out0_hbm, out1_hbm,
                src_ref, dst_ref, w_ref, rows_ref, acc_sh, gsem, ssem):
    c = lax.axis_index("c")
    s = lax.axis_index("s")
    wid = c * 16 + s

    def zrow(i, carry):
        rows_ref[0, i, :] = jnp.zeros((_H,), jnp.float32)
        return carry

    lax.fori_loop(0, _NRS, zrow, 0)
    pltpu.sync_copy(rows_ref.at[0, pl.ds(0, _NRS)],
                    acc_sh.at[pl.ds(s * _NRS, _NRS)])
    plsc.subcore_barrier()

    def stage_and_gather(ci, b):
        e0 = wid * _PW + ci * _CH
        r0 = wid * (_PW // 128) + ci * _RB
        pltpu.sync_copy(src_hbm.at[pl.ds(r0, _RB)], src_ref.at[b])
        pltpu.sync_copy(dst_hbm.at[pl.ds(r0, _RB)], dst_ref.at[b])
        pltpu.sync_copy(w_hbm.at[pl.ds(e0, _CH)], w_ref.at[b])
        return [
            pltpu.async_copy(
                tab_hbm.at[src_ref.at[b, j]],
                rows_ref.at[b, pl.ds(j * 128, 128)],
                gsem.at[b],
            )
            for j in range(_RB)
        ]

    gather_cps = {0: stage_and_gather(0, 0)}
    scatter_cps = {}
    for ci in range(_NCH):
        b = ci % 2
        for cp in gather_cps.pop(ci):
            cp.wait()

        def scale(i, carry2, b=b):
            e = i * 16
            for k in range(16):
                idx = jnp.full((16,), e + k, jnp.int32)
                ws = plsc.load_gather(w_ref.at[b], [idx])
                rows_ref[b, e + k, :] = rows_ref[b, e + k, :] * ws
            return carry2

        lax.fori_loop(0, _CH // 16, scale, 0)
        scatter_cps[ci] = [
            pltpu.async_copy(
                rows_ref.at[b, pl.ds(j * 128, 128)],
                acc_sh.at[dst_ref.at[b, j]],
                ssem.at[b],
                add=True,
            )
            for j in range(_RB)
        ]
        if ci + 1 < _NCH:
            if ci - 1 >= 0:
                for cp in scatter_cps.pop(ci - 1):
                    cp.wait()
            gather_cps[ci + 1] = stage_and_gather(ci + 1, 1 - b)
    for ci in sorted(scatter_cps):
        for cp in scatter_cps[ci]:
            cp.wait()
    plsc.subcore_barrier()

    @pl.when(c == 0)
    def _():
        pltpu.sync_copy(acc_sh.at[pl.ds(s * _NRS, _NRS)],
                        out0_hbm.at[pl.ds(s * _NRS, _NRS)])

    @pl.when(c == 1)
    def _():
        pltpu.sync_copy(acc_sh.at[pl.ds(s * _NRS, _NRS)],
                        out1_hbm.at[pl.ds(s * _NRS, _NRS)])


_BN = 2000


def _tcb_body(deg_ref, x_ref, w1_ref, hp_ref, dinv_ref):
    deg = deg_ref[:, 0:1] + deg_ref[:, 1:2] + 1.0
    dinv = jnp.where(deg > 0, lax.rsqrt(jnp.maximum(deg, 1e-12)), 0.0)
    h = jnp.dot(x_ref[:, :], w1_ref[:, :], preferred_element_type=jnp.float32)
    hp_ref[:, :] = h * dinv
    dinv_ref[:, :] = dinv


_tc_b = pl.pallas_call(
    _tcb_body,
    grid=(_N // _BN,),
    in_specs=[
        pl.BlockSpec((_BN, 2), lambda i: (i, 0)),
        pl.BlockSpec((_BN, _D), lambda i: (i, 0)),
        pl.BlockSpec((_D, _H), lambda i: (0, 0)),
    ],
    out_specs=[
        pl.BlockSpec((_BN, _H), lambda i: (i, 0)),
        pl.BlockSpec((_BN, 1), lambda i: (i, 0)),
    ],
    out_shape=[
        jax.ShapeDtypeStruct((_N, _H), jnp.float32),
        jax.ShapeDtypeStruct((_N, 1), jnp.float32),
    ],
)


def _tcd_body(a0_ref, a1_ref, hp_ref, dv_ref, b1_ref, w2_ref, x1_ref, gp_ref):
    dinv = dv_ref[:, :]
    x1 = dinv * (a0_ref[:, :] + a1_ref[:, :] + hp_ref[:, :]) + b1_ref[:, :]
    x1_ref[:, :] = x1
    r = jnp.maximum(x1, 0.0)
    g = jnp.dot(r, w2_ref[:, :], preferred_element_type=jnp.float32)
    gp_ref[:, :] = g * dinv


_tc_d = pl.pallas_call(
    _tcd_body,
    grid=(_N // _BN,),
    in_specs=[
        pl.BlockSpec((_BN, _H), lambda i: (i, 0)),
        pl.BlockSpec((_BN, _H), lambda i: (i, 0)),
        pl.BlockSpec((_BN, _H), lambda i: (i, 0)),
        pl.BlockSpec((_BN, 1), lambda i: (i, 0)),
        pl.BlockSpec((1, _H), lambda i: (0, 0)),
        pl.BlockSpec((_H, _H), lambda i: (0, 0)),
    ],
    out_specs=[
        pl.BlockSpec((_BN, _H), lambda i: (i, 0)),
        pl.BlockSpec((_BN, _H), lambda i: (i, 0)),
    ],
    out_shape=[
        jax.ShapeDtypeStruct((_N, _H), jnp.float32),
        jax.ShapeDtypeStruct((_N, _H), jnp.float32),
    ],
)


def _tcf_body(a0_ref, a1_ref, gp_ref, dv_ref, b2_ref, out_ref):
    o = dv_ref[:, :] * (a0_ref[:, :] + a1_ref[:, :] + gp_ref[:, :]) + b2_ref[:, :]
    mask = lax.broadcasted_iota(jnp.int32, (_BN, _H), 1) < _C
    z = jnp.where(mask, o, -3.0e38)
    m = jnp.max(z, axis=1, keepdims=True)
    e = jnp.where(mask, jnp.exp(z - m), 0.0)
    lse = jnp.log(jnp.sum(e, axis=1, keepdims=True)) + m
    out_ref[:, :] = lax.slice(o - lse, (0, 0), (_BN, _C))


_tc_f = pl.pallas_call(
    _tcf_body,
    grid=(_N // _BN,),
    in_specs=[
        pl.BlockSpec((_BN, _H), lambda i: (i, 0)),
        pl.BlockSpec((_BN, _H), lambda i: (i, 0)),
        pl.BlockSpec((_BN, _H), lambda i: (i, 0)),
        pl.BlockSpec((_BN, 1), lambda i: (i, 0)),
        pl.BlockSpec((1, _H), lambda i: (0, 0)),
    ],
    out_specs=pl.BlockSpec((_BN, _C), lambda i: (i, 0)),
    out_shape=jax.ShapeDtypeStruct((_N, _C), jnp.float32),
)


def kernel(x, edge_index, edge_weight, W1, b1, W2, b2):
    src = edge_index[0].astype(jnp.int32)
    dst = edge_index[1].astype(jnp.int32)
    w = edge_weight.astype(jnp.float32)

    pad = _EP - _E
    fill = jnp.asarray((np.arange(pad, dtype=np.int32) * 13) % _N)
    srcp = jnp.concatenate([src, fill]).reshape(_EP // 128, 128)
    dstp = jnp.concatenate([dst, fill]).reshape(_EP // 128, 128)
    wp = jnp.concatenate([w, jnp.zeros((pad,), jnp.float32)])

    degp = _deg_kernel(dstp, wp)
    deg2 = jnp.stack([degp[:_N], degp[_NP:_NP + _N]], axis=1)

    hp, dinv = _tc_b(deg2, x, W1)

    a0, a1 = _agg_kernel(hp, srcp, dstp, wp)

    W2p = jnp.zeros((_H, _H), jnp.float32).at[:, :_C].set(W2)
    x1, gp = _tc_d(a0, a1, hp, dinv, b1.reshape(1, _H), W2p)

    c0, c1 = _agg_kernel(gp, srcp, dstp, wp)

    b2p = jnp.zeros((1, _H), jnp.float32).at[0, :_C].set(b2)
    out = _tc_f(c0, c1, gp, dinv, b2p)
    return (out, x1)


# speedup vs baseline: 1.4079x; 1.4079x over previous
import functools

import numpy as np

import jax
import jax.numpy as jnp
from jax import lax
from jax.experimental import pallas as pl
from jax.experimental.pallas import tpu as pltpu
from jax.experimental.pallas import tpu_sc as plsc

_N = 10000
_E = 320000
_D = 128
_H = 16
_C = 10

_NW = 32
_CH = 1024
_RB = _CH // 128
_PW = 10240
_EP = _NW * _PW
_NCH = _PW // _CH
_NP = 10240
_NPS = _NP // 16

_mesh = plsc.VectorSubcoreMesh(core_axis_name="c", subcore_axis_name="s")


@functools.partial(
    pl.kernel,
    out_type=jax.ShapeDtypeStruct((2 * _NP,), jnp.float32),
    scratch_types=[
        pltpu.VMEM((_RB, 128), jnp.int32),
        pltpu.VMEM((_CH,), jnp.float32),
        pltpu.VMEM((_NPS,), jnp.float32),
        pltpu.VMEM_SHARED((_NP,), jnp.float32),
    ],
    mesh=_mesh,
    compiler_params=pltpu.CompilerParams(use_tc_tiling_on_sc=False),
)
def _deg_kernel(dst_hbm, w_hbm, out_hbm, dst_ref, w_ref, zb, deg_sh):
    c = lax.axis_index("c")
    s = lax.axis_index("s")
    wid = c * 16 + s

    def zrow(i, carry):
        zb[pl.ds(i * 16, 16)] = jnp.zeros((16,), jnp.float32)
        return carry

    lax.fori_loop(0, _NPS // 16, zrow, 0)
    pltpu.sync_copy(zb, deg_sh.at[pl.ds(s * _NPS, _NPS)])
    plsc.subcore_barrier()

    def chunk(ci, carry):
        e0 = wid * _PW + ci * _CH
        r0 = wid * (_PW // 128) + ci * _RB
        pltpu.sync_copy(dst_hbm.at[pl.ds(r0, _RB)], dst_ref)
        pltpu.sync_copy(w_hbm.at[pl.ds(e0, _CH)], w_ref)
        for j in range(_RB):
            pltpu.sync_copy(
                w_ref.at[pl.ds(j * 128, 128)],
                deg_sh.at[dst_ref.at[j]],
                add=True,
            )
        return carry

    lax.fori_loop(0, _NCH, chunk, 0)
    plsc.subcore_barrier()
    pltpu.sync_copy(
        deg_sh.at[pl.ds(s * _NPS, _NPS)],
        out_hbm.at[pl.ds(c * _NP + s * _NPS, _NPS)],
    )


_NRS = _N // 16


@functools.partial(
    pl.kernel,
    out_type=[
        jax.ShapeDtypeStruct((_N, _H), jnp.float32),
        jax.ShapeDtypeStruct((_N, _H), jnp.float32),
    ],
    scratch_types=[
        pltpu.VMEM((2, _RB, 128), jnp.int32),
        pltpu.VMEM((2, _RB, 128), jnp.int32),
        pltpu.VMEM((2, _CH), jnp.float32),
        pltpu.VMEM((2, _CH, _H), jnp.float32),
        pltpu.VMEM_SHARED((_N, _H), jnp.float32),
        pltpu.SemaphoreType.DMA((2,)),
        pltpu.SemaphoreType.DMA((2,)),
    ],
    mesh=_mesh,
    compiler_params=pltpu.CompilerParams(
        use_tc_tiling_on_sc=False, needs_layout_passes=False),
)
def _agg_kernel(tab_hbm, src_hbm, dst_hbm, w_hbm, out0_hbm, out1_hbm,
                src_ref, dst_ref, w_ref, rows_ref, acc_sh, gsem, ssem):
    c = lax.axis_index("c")
    s = lax.axis_index("s")
    wid = c * 16 + s

    def zrow(i, carry):
        rows_ref[0, i, :] = jnp.zeros((_H,), jnp.float32)
        return carry

    lax.fori_loop(0, _NRS, zrow, 0)
    pltpu.sync_copy(rows_ref.at[0, pl.ds(0, _NRS)],
                    acc_sh.at[pl.ds(s * _NRS, _NRS)])
    plsc.subcore_barrier()

    def stage_and_gather(ci, b):
        e0 = wid * _PW + ci * _CH
        r0 = wid * (_PW // 128) + ci * _RB
        pltpu.sync_copy(src_hbm.at[pl.ds(r0, _RB)], src_ref.at[b])
        pltpu.sync_copy(dst_hbm.at[pl.ds(r0, _RB)], dst_ref.at[b])
        pltpu.sync_copy(w_hbm.at[pl.ds(e0, _CH)], w_ref.at[b])
        return [
            pltpu.async_copy(
                tab_hbm.at[src_ref.at[b, j]],
                rows_ref.at[b, pl.ds(j * 128, 128)],
                gsem.at[b],
            )
            for j in range(_RB)
        ]

    gather_cps = {0: stage_and_gather(0, 0)}
    scatter_cps = {}
    for ci in range(_NCH):
        b = ci % 2
        for cp in gather_cps.pop(ci):
            cp.wait()

        def scale(i, carry2, b=b):
            e = i * 16
            wv = w_ref[b, pl.ds(e, 16)]
            for k in range(16):
                rows_ref[b, e + k, :] = rows_ref[b, e + k, :] * wv[k]
            return carry2

        lax.fori_loop(0, _CH // 16, scale, 0)
        scatter_cps[ci] = [
            pltpu.async_copy(
                rows_ref.at[b, pl.ds(j * 128, 128)],
                acc_sh.at[dst_ref.at[b, j]],
                ssem.at[b],
                add=True,
            )
            for j in range(_RB)
        ]
        if ci + 1 < _NCH:
            if ci - 1 >= 0:
                for cp in scatter_cps.pop(ci - 1):
                    cp.wait()
            gather_cps[ci + 1] = stage_and_gather(ci + 1, 1 - b)
    for ci in sorted(scatter_cps):
        for cp in scatter_cps[ci]:
            cp.wait()
    plsc.subcore_barrier()

    @pl.when(c == 0)
    def _():
        pltpu.sync_copy(acc_sh.at[pl.ds(s * _NRS, _NRS)],
                        out0_hbm.at[pl.ds(s * _NRS, _NRS)])

    @pl.when(c == 1)
    def _():
        pltpu.sync_copy(acc_sh.at[pl.ds(s * _NRS, _NRS)],
                        out1_hbm.at[pl.ds(s * _NRS, _NRS)])


_BN = 2000


def _tcb_body(deg_ref, x_ref, w1_ref, hp_ref, dinv_ref):
    deg = deg_ref[:, 0:1] + deg_ref[:, 1:2] + 1.0
    dinv = jnp.where(deg > 0, lax.rsqrt(jnp.maximum(deg, 1e-12)), 0.0)
    h = jnp.dot(x_ref[:, :], w1_ref[:, :], preferred_element_type=jnp.float32)
    hp_ref[:, :] = h * dinv
    dinv_ref[:, :] = dinv


_tc_b = pl.pallas_call(
    _tcb_body,
    grid=(_N // _BN,),
    in_specs=[
        pl.BlockSpec((_BN, 2), lambda i: (i, 0)),
        pl.BlockSpec((_BN, _D), lambda i: (i, 0)),
        pl.BlockSpec((_D, _H), lambda i: (0, 0)),
    ],
    out_specs=[
        pl.BlockSpec((_BN, _H), lambda i: (i, 0)),
        pl.BlockSpec((_BN, 1), lambda i: (i, 0)),
    ],
    out_shape=[
        jax.ShapeDtypeStruct((_N, _H), jnp.float32),
        jax.ShapeDtypeStruct((_N, 1), jnp.float32),
    ],
)


def _tcd_body(a0_ref, a1_ref, hp_ref, dv_ref, b1_ref, w2_ref, x1_ref, gp_ref):
    dinv = dv_ref[:, :]
    x1 = dinv * (a0_ref[:, :] + a1_ref[:, :] + hp_ref[:, :]) + b1_ref[:, :]
    x1_ref[:, :] = x1
    r = jnp.maximum(x1, 0.0)
    g = jnp.dot(r, w2_ref[:, :], preferred_element_type=jnp.float32)
    gp_ref[:, :] = g * dinv


_tc_d = pl.pallas_call(
    _tcd_body,
    grid=(_N // _BN,),
    in_specs=[
        pl.BlockSpec((_BN, _H), lambda i: (i, 0)),
        pl.BlockSpec((_BN, _H), lambda i: (i, 0)),
        pl.BlockSpec((_BN, _H), lambda i: (i, 0)),
        pl.BlockSpec((_BN, 1), lambda i: (i, 0)),
        pl.BlockSpec((1, _H), lambda i: (0, 0)),
        pl.BlockSpec((_H, _H), lambda i: (0, 0)),
    ],
    out_specs=[
        pl.BlockSpec((_BN, _H), lambda i: (i, 0)),
        pl.BlockSpec((_BN, _H), lambda i: (i, 0)),
    ],
    out_shape=[
        jax.ShapeDtypeStruct((_N, _H), jnp.float32),
        jax.ShapeDtypeStruct((_N, _H), jnp.float32),
    ],
)


def _tcf_body(a0_ref, a1_ref, gp_ref, dv_ref, b2_ref, out_ref):
    o = dv_ref[:, :] * (a0_ref[:, :] + a1_ref[:, :] + gp_ref[:, :]) + b2_ref[:, :]
    mask = lax.broadcasted_iota(jnp.int32, (_BN, _H), 1) < _C
    z = jnp.where(mask, o, -3.0e38)
    m = jnp.max(z, axis=1, keepdims=True)
    e = jnp.where(mask, jnp.exp(z - m), 0.0)
    lse = jnp.log(jnp.sum(e, axis=1, keepdims=True)) + m
    out_ref[:, :] = lax.slice(o - lse, (0, 0), (_BN, _C))


_tc_f = pl.pallas_call(
    _tcf_body,
    grid=(_N // _BN,),
    in_specs=[
        pl.BlockSpec((_BN, _H), lambda i: (i, 0)),
        pl.BlockSpec((_BN, _H), lambda i: (i, 0)),
        pl.BlockSpec((_BN, _H), lambda i: (i, 0)),
        pl.BlockSpec((_BN, 1), lambda i: (i, 0)),
        pl.BlockSpec((1, _H), lambda i: (0, 0)),
    ],
    out_specs=pl.BlockSpec((_BN, _C), lambda i: (i, 0)),
    out_shape=jax.ShapeDtypeStruct((_N, _C), jnp.float32),
)


def kernel(x, edge_index, edge_weight, W1, b1, W2, b2):
    src = edge_index[0].astype(jnp.int32)
    dst = edge_index[1].astype(jnp.int32)
    w = edge_weight.astype(jnp.float32)

    pad = _EP - _E
    fill = jnp.asarray((np.arange(pad, dtype=np.int32) * 13) % _N)
    srcp = jnp.concatenate([src, fill]).reshape(_EP // 128, 128)
    dstp = jnp.concatenate([dst, fill]).reshape(_EP // 128, 128)
    wp = jnp.concatenate([w, jnp.zeros((pad,), jnp.float32)])

    degp = _deg_kernel(dstp, wp)
    deg2 = jnp.stack([degp[:_N], degp[_NP:_NP + _N]], axis=1)

    hp, dinv = _tc_b(deg2, x, W1)

    a0, a1 = _agg_kernel(hp, srcp, dstp, wp)

    W2p = jnp.zeros((_H, _H), jnp.float32).at[:, :_C].set(W2)
    x1, gp = _tc_d(a0, a1, hp, dinv, b1.reshape(1, _H), W2p)

    c0, c1 = _agg_kernel(gp, srcp, dstp, wp)

    b2p = jnp.zeros((1, _H), jnp.float32).at[0, :_C].set(b2)
    out = _tc_f(c0, c1, gp, dinv, b2p)
    return (out, x1)


# speedup vs baseline: 1.4358x; 1.0198x over previous
import functools

import numpy as np

import jax
import jax.numpy as jnp
from jax import lax
from jax.experimental import pallas as pl
from jax.experimental.pallas import tpu as pltpu
from jax.experimental.pallas import tpu_sc as plsc

_N = 10000
_E = 320000
_D = 128
_H = 16
_C = 10

_NW = 32
_CH = 1024
_RB = _CH // 128
_PW = 10240
_EP = _NW * _PW
_NCH = _PW // _CH
_NP = 10240
_NPS = _NP // 16

_mesh = plsc.VectorSubcoreMesh(core_axis_name="c", subcore_axis_name="s")


@functools.partial(
    pl.kernel,
    out_type=jax.ShapeDtypeStruct((2 * _NP,), jnp.float32),
    scratch_types=[
        pltpu.VMEM((_RB, 128), jnp.int32),
        pltpu.VMEM((_CH,), jnp.float32),
        pltpu.VMEM((_NPS,), jnp.float32),
        pltpu.VMEM_SHARED((_NP,), jnp.float32),
    ],
    mesh=_mesh,
    compiler_params=pltpu.CompilerParams(use_tc_tiling_on_sc=False),
)
def _deg_kernel(dst_hbm, w_hbm, out_hbm, dst_ref, w_ref, zb, deg_sh):
    c = lax.axis_index("c")
    s = lax.axis_index("s")
    wid = c * 16 + s

    def zrow(i, carry):
        zb[pl.ds(i * 16, 16)] = jnp.zeros((16,), jnp.float32)
        return carry

    lax.fori_loop(0, _NPS // 16, zrow, 0)
    pltpu.sync_copy(zb, deg_sh.at[pl.ds(s * _NPS, _NPS)])
    plsc.subcore_barrier()

    def chunk(ci, carry):
        e0 = wid * _PW + ci * _CH
        r0 = wid * (_PW // 128) + ci * _RB
        pltpu.sync_copy(dst_hbm.at[pl.ds(r0, _RB)], dst_ref)
        pltpu.sync_copy(w_hbm.at[pl.ds(e0, _CH)], w_ref)
        for j in range(_RB):
            pltpu.sync_copy(
                w_ref.at[pl.ds(j * 128, 128)],
                deg_sh.at[dst_ref.at[j]],
                add=True,
            )
        return carry

    lax.fori_loop(0, _NCH, chunk, 0)
    plsc.subcore_barrier()
    pltpu.sync_copy(
        deg_sh.at[pl.ds(s * _NPS, _NPS)],
        out_hbm.at[pl.ds(c * _NP + s * _NPS, _NPS)],
    )


_NRS = _N // 16


@functools.partial(
    pl.kernel,
    out_type=[
        jax.ShapeDtypeStruct((_N, _H), jnp.float32),
        jax.ShapeDtypeStruct((_N, _H), jnp.float32),
    ],
    scratch_types=[
        pltpu.VMEM((2, _RB, 128), jnp.int32),
        pltpu.VMEM((2, _RB, 128), jnp.int32),
        pltpu.VMEM((2, _CH), jnp.float32),
        pltpu.VMEM((2, _CH, _H), jnp.float32),
        pltpu.VMEM_SHARED((_N, _H), jnp.float32),
        pltpu.SemaphoreType.DMA((2,)),
        pltpu.SemaphoreType.DMA((2,)),
    ],
    mesh=_mesh,
    compiler_params=pltpu.CompilerParams(
        use_tc_tiling_on_sc=False, needs_layout_passes=False),
)
def _agg_kernel(tab_hbm, src_hbm, dst_hbm, w_hbm, out0_hbm, out1_hbm,
                src_ref, dst_ref, w_ref, rows_ref, acc_sh, gsem, ssem):
    c = lax.axis_index("c")
    s = lax.axis_index("s")
    wid = c * 16 + s

    def zrow(i, carry):
        rows_ref[0, i, :] = jnp.zeros((_H,), jnp.float32)
        return carry

    lax.fori_loop(0, _NRS, zrow, 0)
    pltpu.sync_copy(rows_ref.at[0, pl.ds(0, _NRS)],
                    acc_sh.at[pl.ds(s * _NRS, _NRS)])
    plsc.subcore_barrier()

    def stage_and_gather(ci, b):
        e0 = wid * _PW + ci * _CH
        r0 = wid * (_PW // 128) + ci * _RB
        pltpu.sync_copy(src_hbm.at[pl.ds(r0, _RB)], src_ref.at[b])
        pltpu.sync_copy(dst_hbm.at[pl.ds(r0, _RB)], dst_ref.at[b])
        pltpu.sync_copy(w_hbm.at[pl.ds(e0, _CH)], w_ref.at[b])
        return [
            pltpu.async_copy(
                tab_hbm.at[src_ref.at[b, j]],
                rows_ref.at[b, pl.ds(j * 128, 128)],
                gsem.at[b],
            )
            for j in range(_RB)
        ]

    gather_cps = {0: stage_and_gather(0, 0)}
    scatter_cps = {}
    for ci in range(_NCH):
        b = ci % 2
        for cp in gather_cps.pop(ci):
            cp.wait()

        @plsc.parallel_loop(0, _CH, step=16, unroll=2)
        def _scale(e, b=b):
            wv = w_ref[b, pl.ds(e, 16)]
            for k in range(16):
                rows_ref[b, e + k, :] = rows_ref[b, e + k, :] * wv[k]
        scatter_cps[ci] = [
            pltpu.async_copy(
                rows_ref.at[b, pl.ds(j * 128, 128)],
                acc_sh.at[dst_ref.at[b, j]],
                ssem.at[b],
                add=True,
            )
            for j in range(_RB)
        ]
        if ci + 1 < _NCH:
            if ci - 1 >= 0:
                for cp in scatter_cps.pop(ci - 1):
                    cp.wait()
            gather_cps[ci + 1] = stage_and_gather(ci + 1, 1 - b)
    for ci in sorted(scatter_cps):
        for cp in scatter_cps[ci]:
            cp.wait()
    plsc.subcore_barrier()

    @pl.when(c == 0)
    def _():
        pltpu.sync_copy(acc_sh.at[pl.ds(s * _NRS, _NRS)],
                        out0_hbm.at[pl.ds(s * _NRS, _NRS)])

    @pl.when(c == 1)
    def _():
        pltpu.sync_copy(acc_sh.at[pl.ds(s * _NRS, _NRS)],
                        out1_hbm.at[pl.ds(s * _NRS, _NRS)])


_BN = 2000


def _tcb_body(deg_ref, x_ref, w1_ref, hp_ref, dinv_ref):
    deg = deg_ref[:, 0:1] + deg_ref[:, 1:2] + 1.0
    dinv = jnp.where(deg > 0, lax.rsqrt(jnp.maximum(deg, 1e-12)), 0.0)
    h = jnp.dot(x_ref[:, :], w1_ref[:, :], preferred_element_type=jnp.float32)
    hp_ref[:, :] = h * dinv
    dinv_ref[:, :] = dinv


_tc_b = pl.pallas_call(
    _tcb_body,
    grid=(_N // _BN,),
    in_specs=[
        pl.BlockSpec((_BN, 2), lambda i: (i, 0)),
        pl.BlockSpec((_BN, _D), lambda i: (i, 0)),
        pl.BlockSpec((_D, _H), lambda i: (0, 0)),
    ],
    out_specs=[
        pl.BlockSpec((_BN, _H), lambda i: (i, 0)),
        pl.BlockSpec((_BN, 1), lambda i: (i, 0)),
    ],
    out_shape=[
        jax.ShapeDtypeStruct((_N, _H), jnp.float32),
        jax.ShapeDtypeStruct((_N, 1), jnp.float32),
    ],
)


def _tcd_body(a0_ref, a1_ref, hp_ref, dv_ref, b1_ref, w2_ref, x1_ref, gp_ref):
    dinv = dv_ref[:, :]
    x1 = dinv * (a0_ref[:, :] + a1_ref[:, :] + hp_ref[:, :]) + b1_ref[:, :]
    x1_ref[:, :] = x1
    r = jnp.maximum(x1, 0.0)
    g = jnp.dot(r, w2_ref[:, :], preferred_element_type=jnp.float32)
    gp_ref[:, :] = g * dinv


_tc_d = pl.pallas_call(
    _tcd_body,
    grid=(_N // _BN,),
    in_specs=[
        pl.BlockSpec((_BN, _H), lambda i: (i, 0)),
        pl.BlockSpec((_BN, _H), lambda i: (i, 0)),
        pl.BlockSpec((_BN, _H), lambda i: (i, 0)),
        pl.BlockSpec((_BN, 1), lambda i: (i, 0)),
        pl.BlockSpec((1, _H), lambda i: (0, 0)),
        pl.BlockSpec((_H, _H), lambda i: (0, 0)),
    ],
    out_specs=[
        pl.BlockSpec((_BN, _H), lambda i: (i, 0)),
        pl.BlockSpec((_BN, _H), lambda i: (i, 0)),
    ],
    out_shape=[
        jax.ShapeDtypeStruct((_N, _H), jnp.float32),
        jax.ShapeDtypeStruct((_N, _H), jnp.float32),
    ],
)


def _tcf_body(a0_ref, a1_ref, gp_ref, dv_ref, b2_ref, out_ref):
    o = dv_ref[:, :] * (a0_ref[:, :] + a1_ref[:, :] + gp_ref[:, :]) + b2_ref[:, :]
    mask = lax.broadcasted_iota(jnp.int32, (_BN, _H), 1) < _C
    z = jnp.where(mask, o, -3.0e38)
    m = jnp.max(z, axis=1, keepdims=True)
    e = jnp.where(mask, jnp.exp(z - m), 0.0)
    lse = jnp.log(jnp.sum(e, axis=1, keepdims=True)) + m
    out_ref[:, :] = lax.slice(o - lse, (0, 0), (_BN, _C))


_tc_f = pl.pallas_call(
    _tcf_body,
    grid=(_N // _BN,),
    in_specs=[
        pl.BlockSpec((_BN, _H), lambda i: (i, 0)),
        pl.BlockSpec((_BN, _H), lambda i: (i, 0)),
        pl.BlockSpec((_BN, _H), lambda i: (i, 0)),
        pl.BlockSpec((_BN, 1), lambda i: (i, 0)),
        pl.BlockSpec((1, _H), lambda i: (0, 0)),
    ],
    out_specs=pl.BlockSpec((_BN, _C), lambda i: (i, 0)),
    out_shape=jax.ShapeDtypeStruct((_N, _C), jnp.float32),
)


def kernel(x, edge_index, edge_weight, W1, b1, W2, b2):
    src = edge_index[0].astype(jnp.int32)
    dst = edge_index[1].astype(jnp.int32)
    w = edge_weight.astype(jnp.float32)

    pad = _EP - _E
    fill = jnp.asarray((np.arange(pad, dtype=np.int32) * 13) % _N)
    srcp = jnp.concatenate([src, fill]).reshape(_EP // 128, 128)
    dstp = jnp.concatenate([dst, fill]).reshape(_EP // 128, 128)
    wp = jnp.concatenate([w, jnp.zeros((pad,), jnp.float32)])

    degp = _deg_kernel(dstp, wp)
    deg2 = jnp.stack([degp[:_N], degp[_NP:_NP + _N]], axis=1)

    hp, dinv = _tc_b(deg2, x, W1)

    a0, a1 = _agg_kernel(hp, srcp, dstp, wp)

    W2p = jnp.zeros((_H, _H), jnp.float32).at[:, :_C].set(W2)
    x1, gp = _tc_d(a0, a1, hp, dinv, b1.reshape(1, _H), W2p)

    c0, c1 = _agg_kernel(gp, srcp, dstp, wp)

    b2p = jnp.zeros((1, _H), jnp.float32).at[0, :_C].set(b2)
    out = _tc_f(c0, c1, gp, dinv, b2p)
    return (out, x1)


# speedup vs baseline: 1.5166x; 1.0563x over previous
import functools

import numpy as np

import jax
import jax.numpy as jnp
from jax import lax
from jax.experimental import pallas as pl
from jax.experimental.pallas import tpu as pltpu
from jax.experimental.pallas import tpu_sc as plsc

_N = 10000
_E = 320000
_D = 128
_H = 16
_C = 10

_NW = 32
_CH = 1024
_RB = _CH // 128
_PW = 10240
_EP = _NW * _PW
_NCH = _PW // _CH
_NP = 10240
_NPS = _NP // 16

_mesh = plsc.VectorSubcoreMesh(core_axis_name="c", subcore_axis_name="s")


@functools.partial(
    pl.kernel,
    out_type=jax.ShapeDtypeStruct((2 * _NP,), jnp.float32),
    scratch_types=[
        pltpu.VMEM((_RB, 128), jnp.int32),
        pltpu.VMEM((_CH,), jnp.float32),
        pltpu.VMEM((_NPS,), jnp.float32),
        pltpu.VMEM_SHARED((_NP,), jnp.float32),
    ],
    mesh=_mesh,
    compiler_params=pltpu.CompilerParams(use_tc_tiling_on_sc=False),
)
def _deg_kernel(dst_hbm, w_hbm, out_hbm, dst_ref, w_ref, zb, deg_sh):
    c = lax.axis_index("c")
    s = lax.axis_index("s")
    wid = c * 16 + s

    def zrow(i, carry):
        zb[pl.ds(i * 16, 16)] = jnp.zeros((16,), jnp.float32)
        return carry

    lax.fori_loop(0, _NPS // 16, zrow, 0)
    pltpu.sync_copy(zb, deg_sh.at[pl.ds(s * _NPS, _NPS)])
    plsc.subcore_barrier()

    def chunk(ci, carry):
        e0 = wid * _PW + ci * _CH
        r0 = wid * (_PW // 128) + ci * _RB
        pltpu.sync_copy(dst_hbm.at[pl.ds(r0, _RB)], dst_ref)
        pltpu.sync_copy(w_hbm.at[pl.ds(e0, _CH)], w_ref)
        for j in range(_RB):
            pltpu.sync_copy(
                w_ref.at[pl.ds(j * 128, 128)],
                deg_sh.at[dst_ref.at[j]],
                add=True,
            )
        return carry

    lax.fori_loop(0, _NCH, chunk, 0)
    plsc.subcore_barrier()
    pltpu.sync_copy(
        deg_sh.at[pl.ds(s * _NPS, _NPS)],
        out_hbm.at[pl.ds(c * _NP + s * _NPS, _NPS)],
    )


_NRS = _N // 16


@functools.partial(
    pl.kernel,
    out_type=[
        jax.ShapeDtypeStruct((_N, _H), jnp.float32),
        jax.ShapeDtypeStruct((_N, _H), jnp.float32),
    ],
    scratch_types=[
        pltpu.VMEM((2, _RB, 128), jnp.int32),
        pltpu.VMEM((2, _RB, 128), jnp.int32),
        pltpu.VMEM((2, _CH), jnp.float32),
        pltpu.VMEM((2, _CH, _H), jnp.float32),
        pltpu.VMEM_SHARED((_N, _H), jnp.float32),
        pltpu.SemaphoreType.DMA((2,)),
        pltpu.SemaphoreType.DMA((2,)),
    ],
    mesh=_mesh,
    compiler_params=pltpu.CompilerParams(
        use_tc_tiling_on_sc=False, needs_layout_passes=False),
)
def _agg_kernel(tab_hbm, src_hbm, dst_hbm, w_hbm, out0_hbm, out1_hbm,
                src_ref, dst_ref, w_ref, rows_ref, acc_sh, gsem, ssem):
    c = lax.axis_index("c")
    s = lax.axis_index("s")
    wid = c * 16 + s

    def zrow(i, carry):
        rows_ref[0, i, :] = jnp.zeros((_H,), jnp.float32)
        return carry

    lax.fori_loop(0, _NRS, zrow, 0)
    pltpu.sync_copy(rows_ref.at[0, pl.ds(0, _NRS)],
                    acc_sh.at[pl.ds(s * _NRS, _NRS)])
    plsc.subcore_barrier()

    def stage_and_gather(ci, b):
        e0 = wid * _PW + ci * _CH
        r0 = wid * (_PW // 128) + ci * _RB
        pltpu.sync_copy(src_hbm.at[pl.ds(r0, _RB)], src_ref.at[b])
        pltpu.sync_copy(dst_hbm.at[pl.ds(r0, _RB)], dst_ref.at[b])
        pltpu.sync_copy(w_hbm.at[pl.ds(e0, _CH)], w_ref.at[b])
        return [
            pltpu.async_copy(
                tab_hbm.at[src_ref.at[b, j]],
                rows_ref.at[b, pl.ds(j * 128, 128)],
                gsem.at[b],
            )
            for j in range(_RB)
        ]

    gather_cps = {0: stage_and_gather(0, 0)}
    scatter_cps = {}
    for ci in range(_NCH):
        b = ci % 2
        for cp in gather_cps.pop(ci):
            cp.wait()
        if ci + 1 < _NCH:
            if ci - 1 >= 0:
                for cp in scatter_cps.pop(ci - 1):
                    cp.wait()
            gather_cps[ci + 1] = stage_and_gather(ci + 1, 1 - b)

        @plsc.parallel_loop(0, _CH, step=16, unroll=2)
        def _scale(e, b=b):
            wv = w_ref[b, pl.ds(e, 16)]
            for k in range(16):
                rows_ref[b, e + k, :] = rows_ref[b, e + k, :] * wv[k]
        scatter_cps[ci] = [
            pltpu.async_copy(
                rows_ref.at[b, pl.ds(j * 128, 128)],
                acc_sh.at[dst_ref.at[b, j]],
                ssem.at[b],
                add=True,
            )
            for j in range(_RB)
        ]
    for ci in sorted(scatter_cps):
        for cp in scatter_cps[ci]:
            cp.wait()
    plsc.subcore_barrier()

    @pl.when(c == 0)
    def _():
        pltpu.sync_copy(acc_sh.at[pl.ds(s * _NRS, _NRS)],
                        out0_hbm.at[pl.ds(s * _NRS, _NRS)])

    @pl.when(c == 1)
    def _():
        pltpu.sync_copy(acc_sh.at[pl.ds(s * _NRS, _NRS)],
                        out1_hbm.at[pl.ds(s * _NRS, _NRS)])


_BN = 2000


def _tcb_body(deg_ref, x_ref, w1_ref, hp_ref, dinv_ref):
    deg = deg_ref[:, 0:1] + deg_ref[:, 1:2] + 1.0
    dinv = jnp.where(deg > 0, lax.rsqrt(jnp.maximum(deg, 1e-12)), 0.0)
    h = jnp.dot(x_ref[:, :], w1_ref[:, :], preferred_element_type=jnp.float32)
    hp_ref[:, :] = h * dinv
    dinv_ref[:, :] = dinv


_tc_b = pl.pallas_call(
    _tcb_body,
    grid=(_N // _BN,),
    in_specs=[
        pl.BlockSpec((_BN, 2), lambda i: (i, 0)),
        pl.BlockSpec((_BN, _D), lambda i: (i, 0)),
        pl.BlockSpec((_D, _H), lambda i: (0, 0)),
    ],
    out_specs=[
        pl.BlockSpec((_BN, _H), lambda i: (i, 0)),
        pl.BlockSpec((_BN, 1), lambda i: (i, 0)),
    ],
    out_shape=[
        jax.ShapeDtypeStruct((_N, _H), jnp.float32),
        jax.ShapeDtypeStruct((_N, 1), jnp.float32),
    ],
)


def _tcd_body(a0_ref, a1_ref, hp_ref, dv_ref, b1_ref, w2_ref, x1_ref, gp_ref):
    dinv = dv_ref[:, :]
    x1 = dinv * (a0_ref[:, :] + a1_ref[:, :] + hp_ref[:, :]) + b1_ref[:, :]
    x1_ref[:, :] = x1
    r = jnp.maximum(x1, 0.0)
    g = jnp.dot(r, w2_ref[:, :], preferred_element_type=jnp.float32)
    gp_ref[:, :] = g * dinv


_tc_d = pl.pallas_call(
    _tcd_body,
    grid=(_N // _BN,),
    in_specs=[
        pl.BlockSpec((_BN, _H), lambda i: (i, 0)),
        pl.BlockSpec((_BN, _H), lambda i: (i, 0)),
        pl.BlockSpec((_BN, _H), lambda i: (i, 0)),
        pl.BlockSpec((_BN, 1), lambda i: (i, 0)),
        pl.BlockSpec((1, _H), lambda i: (0, 0)),
        pl.BlockSpec((_H, _H), lambda i: (0, 0)),
    ],
    out_specs=[
        pl.BlockSpec((_BN, _H), lambda i: (i, 0)),
        pl.BlockSpec((_BN, _H), lambda i: (i, 0)),
    ],
    out_shape=[
        jax.ShapeDtypeStruct((_N, _H), jnp.float32),
        jax.ShapeDtypeStruct((_N, _H), jnp.float32),
    ],
)


def _tcf_body(a0_ref, a1_ref, gp_ref, dv_ref, b2_ref, out_ref):
    o = dv_ref[:, :] * (a0_ref[:, :] + a1_ref[:, :] + gp_ref[:, :]) + b2_ref[:, :]
    mask = lax.broadcasted_iota(jnp.int32, (_BN, _H), 1) < _C
    z = jnp.where(mask, o, -3.0e38)
    m = jnp.max(z, axis=1, keepdims=True)
    e = jnp.where(mask, jnp.exp(z - m), 0.0)
    lse = jnp.log(jnp.sum(e, axis=1, keepdims=True)) + m
    out_ref[:, :] = lax.slice(o - lse, (0, 0), (_BN, _C))


_tc_f = pl.pallas_call(
    _tcf_body,
    grid=(_N // _BN,),
    in_specs=[
        pl.BlockSpec((_BN, _H), lambda i: (i, 0)),
        pl.BlockSpec((_BN, _H), lambda i: (i, 0)),
        pl.BlockSpec((_BN, _H), lambda i: (i, 0)),
        pl.BlockSpec((_BN, 1), lambda i: (i, 0)),
        pl.BlockSpec((1, _H), lambda i: (0, 0)),
    ],
    out_specs=pl.BlockSpec((_BN, _C), lambda i: (i, 0)),
    out_shape=jax.ShapeDtypeStruct((_N, _C), jnp.float32),
)


def kernel(x, edge_index, edge_weight, W1, b1, W2, b2):
    src = edge_index[0].astype(jnp.int32)
    dst = edge_index[1].astype(jnp.int32)
    w = edge_weight.astype(jnp.float32)

    pad = _EP - _E
    fill = jnp.asarray((np.arange(pad, dtype=np.int32) * 13) % _N)
    srcp = jnp.concatenate([src, fill]).reshape(_EP // 128, 128)
    dstp = jnp.concatenate([dst, fill]).reshape(_EP // 128, 128)
    wp = jnp.concatenate([w, jnp.zeros((pad,), jnp.float32)])

    degp = _deg_kernel(dstp, wp)
    deg2 = jnp.stack([degp[:_N], degp[_NP:_NP + _N]], axis=1)

    hp, dinv = _tc_b(deg2, x, W1)

    a0, a1 = _agg_kernel(hp, srcp, dstp, wp)

    W2p = jnp.zeros((_H, _H), jnp.float32).at[:, :_C].set(W2)
    x1, gp = _tc_d(a0, a1, hp, dinv, b1.reshape(1, _H), W2p)

    c0, c1 = _agg_kernel(gp, srcp, dstp, wp)

    b2p = jnp.zeros((1, _H), jnp.float32).at[0, :_C].set(b2)
    out = _tc_f(c0, c1, gp, dinv, b2p)
    return (out, x1)


# speedup vs baseline: 1.5932x; 1.0505x over previous
import functools

import numpy as np

import jax
import jax.numpy as jnp
from jax import lax
from jax.experimental import pallas as pl
from jax.experimental.pallas import tpu as pltpu
from jax.experimental.pallas import tpu_sc as plsc

_N = 10000
_E = 320000
_D = 128
_H = 16
_C = 10

_NW = 32
_CH = 1024
_RB = _CH // 128
_PW = 10240
_EP = _NW * _PW
_NCH = _PW // _CH
_NP = 10240
_NPS = _NP // 16

_mesh = plsc.VectorSubcoreMesh(core_axis_name="c", subcore_axis_name="s")


@functools.partial(
    pl.kernel,
    out_type=jax.ShapeDtypeStruct((2 * _NP,), jnp.float32),
    scratch_types=[
        pltpu.VMEM((2, _RB, 128), jnp.int32),
        pltpu.VMEM((2, _CH), jnp.float32),
        pltpu.VMEM((_NPS,), jnp.float32),
        pltpu.VMEM_SHARED((_NP,), jnp.float32),
        pltpu.SemaphoreType.DMA((2,)),
        pltpu.SemaphoreType.DMA((2,)),
    ],
    mesh=_mesh,
    compiler_params=pltpu.CompilerParams(use_tc_tiling_on_sc=False),
)
def _deg_kernel(dst_hbm, w_hbm, out_hbm, dst_ref, w_ref, zb, deg_sh, gsem, ssem):
    c = lax.axis_index("c")
    s = lax.axis_index("s")
    wid = c * 16 + s

    def zrow(i, carry):
        zb[pl.ds(i * 16, 16)] = jnp.zeros((16,), jnp.float32)
        return carry

    lax.fori_loop(0, _NPS // 16, zrow, 0)
    pltpu.sync_copy(zb, deg_sh.at[pl.ds(s * _NPS, _NPS)])
    plsc.subcore_barrier()

    def stage(ci, b):
        e0 = wid * _PW + ci * _CH
        r0 = wid * (_PW // 128) + ci * _RB
        return [
            pltpu.async_copy(dst_hbm.at[pl.ds(r0, _RB)], dst_ref.at[b],
                             gsem.at[b]),
            pltpu.async_copy(w_hbm.at[pl.ds(e0, _CH)], w_ref.at[b],
                             gsem.at[b]),
        ]

    stage_cps = {0: stage(0, 0)}
    scatter_cps = {}
    for ci in range(_NCH):
        b = ci % 2
        for cp in stage_cps.pop(ci):
            cp.wait()
        if ci + 1 < _NCH:
            if ci - 1 >= 0:
                for cp in scatter_cps.pop(ci - 1):
                    cp.wait()
            stage_cps[ci + 1] = stage(ci + 1, 1 - b)
        scatter_cps[ci] = [
            pltpu.async_copy(
                w_ref.at[b, pl.ds(j * 128, 128)],
                deg_sh.at[dst_ref.at[b, j]],
                ssem.at[b],
                add=True,
            )
            for j in range(_RB)
        ]
    for ci in sorted(scatter_cps):
        for cp in scatter_cps[ci]:
            cp.wait()
    plsc.subcore_barrier()
    pltpu.sync_copy(
        deg_sh.at[pl.ds(s * _NPS, _NPS)],
        out_hbm.at[pl.ds(c * _NP + s * _NPS, _NPS)],
    )


_NRS = _N // 16


@functools.partial(
    pl.kernel,
    out_type=[
        jax.ShapeDtypeStruct((_N, _H), jnp.float32),
        jax.ShapeDtypeStruct((_N, _H), jnp.float32),
    ],
    scratch_types=[
        pltpu.VMEM((2, _RB, 128), jnp.int32),
        pltpu.VMEM((2, _RB, 128), jnp.int32),
        pltpu.VMEM((2, _CH), jnp.float32),
        pltpu.VMEM((2, _CH, _H), jnp.float32),
        pltpu.VMEM_SHARED((_N, _H), jnp.float32),
        pltpu.SemaphoreType.DMA((2,)),
        pltpu.SemaphoreType.DMA((2,)),
    ],
    mesh=_mesh,
    compiler_params=pltpu.CompilerParams(
        use_tc_tiling_on_sc=False, needs_layout_passes=False),
)
def _agg_kernel(tab_hbm, src_hbm, dst_hbm, w_hbm, out0_hbm, out1_hbm,
                src_ref, dst_ref, w_ref, rows_ref, acc_sh, gsem, ssem):
    c = lax.axis_index("c")
    s = lax.axis_index("s")
    wid = c * 16 + s

    def zrow(i, carry):
        rows_ref[0, i, :] = jnp.zeros((_H,), jnp.float32)
        return carry

    lax.fori_loop(0, _NRS, zrow, 0)
    pltpu.sync_copy(rows_ref.at[0, pl.ds(0, _NRS)],
                    acc_sh.at[pl.ds(s * _NRS, _NRS)])
    plsc.subcore_barrier()

    def stage_and_gather(ci, b):
        e0 = wid * _PW + ci * _CH
        r0 = wid * (_PW // 128) + ci * _RB
        pltpu.sync_copy(src_hbm.at[pl.ds(r0, _RB)], src_ref.at[b])
        pltpu.sync_copy(dst_hbm.at[pl.ds(r0, _RB)], dst_ref.at[b])
        pltpu.sync_copy(w_hbm.at[pl.ds(e0, _CH)], w_ref.at[b])
        return [
            pltpu.async_copy(
                tab_hbm.at[src_ref.at[b, j]],
                rows_ref.at[b, pl.ds(j * 128, 128)],
                gsem.at[b],
            )
            for j in range(_RB)
        ]

    gather_cps = {0: stage_and_gather(0, 0)}
    scatter_cps = {}
    for ci in range(_NCH):
        b = ci % 2
        for cp in gather_cps.pop(ci):
            cp.wait()
        if ci + 1 < _NCH:
            if ci - 1 >= 0:
                for cp in scatter_cps.pop(ci - 1):
                    cp.wait()
            gather_cps[ci + 1] = stage_and_gather(ci + 1, 1 - b)

        @plsc.parallel_loop(0, _CH, step=16, unroll=2)
        def _scale(e, b=b):
            wv = w_ref[b, pl.ds(e, 16)]
            for k in range(16):
                rows_ref[b, e + k, :] = rows_ref[b, e + k, :] * wv[k]
        scatter_cps[ci] = [
            pltpu.async_copy(
                rows_ref.at[b, pl.ds(j * 128, 128)],
                acc_sh.at[dst_ref.at[b, j]],
                ssem.at[b],
                add=True,
            )
            for j in range(_RB)
        ]
    for ci in sorted(scatter_cps):
        for cp in scatter_cps[ci]:
            cp.wait()
    plsc.subcore_barrier()

    @pl.when(c == 0)
    def _():
        pltpu.sync_copy(acc_sh.at[pl.ds(s * _NRS, _NRS)],
                        out0_hbm.at[pl.ds(s * _NRS, _NRS)])

    @pl.when(c == 1)
    def _():
        pltpu.sync_copy(acc_sh.at[pl.ds(s * _NRS, _NRS)],
                        out1_hbm.at[pl.ds(s * _NRS, _NRS)])


_BN = 2000


def _dinv_of(deg_ref):
    deg = deg_ref[:, 0:1] + deg_ref[:, 1:2] + 1.0
    return jnp.where(deg > 0, lax.rsqrt(jnp.maximum(deg, 1e-12)), 0.0)


def _tcb_body(deg_ref, x_ref, w1_ref, hp_ref):
    h = jnp.dot(x_ref[:, :], w1_ref[:, :], preferred_element_type=jnp.float32)
    hp_ref[:, :] = h * _dinv_of(deg_ref)


_tc_b = pl.pallas_call(
    _tcb_body,
    grid=(_N // _BN,),
    in_specs=[
        pl.BlockSpec((_BN, 2), lambda i: (i, 0)),
        pl.BlockSpec((_BN, _D), lambda i: (i, 0)),
        pl.BlockSpec((_D, _H), lambda i: (0, 0)),
    ],
    out_specs=pl.BlockSpec((_BN, _H), lambda i: (i, 0)),
    out_shape=jax.ShapeDtypeStruct((_N, _H), jnp.float32),
)


def _tcd_body(deg_ref, a0_ref, a1_ref, hp_ref, b1_ref, w2_ref, x1_ref, gp_ref):
    dinv = _dinv_of(deg_ref)
    x1 = dinv * (a0_ref[:, :] + a1_ref[:, :] + hp_ref[:, :]) + b1_ref[:, :]
    x1_ref[:, :] = x1
    r = jnp.maximum(x1, 0.0)
    g = jnp.dot(r, w2_ref[:, :], preferred_element_type=jnp.float32)
    gp_ref[:, :] = g * dinv


_tc_d = pl.pallas_call(
    _tcd_body,
    grid=(_N // _BN,),
    in_specs=[
        pl.BlockSpec((_BN, 2), lambda i: (i, 0)),
        pl.BlockSpec((_BN, _H), lambda i: (i, 0)),
        pl.BlockSpec((_BN, _H), lambda i: (i, 0)),
        pl.BlockSpec((_BN, _H), lambda i: (i, 0)),
        pl.BlockSpec((1, _H), lambda i: (0, 0)),
        pl.BlockSpec((_H, _H), lambda i: (0, 0)),
    ],
    out_specs=[
        pl.BlockSpec((_BN, _H), lambda i: (i, 0)),
        pl.BlockSpec((_BN, _H), lambda i: (i, 0)),
    ],
    out_shape=[
        jax.ShapeDtypeStruct((_N, _H), jnp.float32),
        jax.ShapeDtypeStruct((_N, _H), jnp.float32),
    ],
)


def _tcf_body(deg_ref, a0_ref, a1_ref, gp_ref, b2_ref, out_ref):
    dinv = _dinv_of(deg_ref)
    o = dinv * (a0_ref[:, :] + a1_ref[:, :] + gp_ref[:, :]) + b2_ref[:, :]
    mask = lax.broadcasted_iota(jnp.int32, (_BN, _H), 1) < _C
    z = jnp.where(mask, o, -3.0e38)
    m = jnp.max(z, axis=1, keepdims=True)
    e = jnp.where(mask, jnp.exp(z - m), 0.0)
    lse = jnp.log(jnp.sum(e, axis=1, keepdims=True)) + m
    out_ref[:, :] = lax.slice(o - lse, (0, 0), (_BN, _C))


_tc_f = pl.pallas_call(
    _tcf_body,
    grid=(_N // _BN,),
    in_specs=[
        pl.BlockSpec((_BN, 2), lambda i: (i, 0)),
        pl.BlockSpec((_BN, _H), lambda i: (i, 0)),
        pl.BlockSpec((_BN, _H), lambda i: (i, 0)),
        pl.BlockSpec((_BN, _H), lambda i: (i, 0)),
        pl.BlockSpec((1, _H), lambda i: (0, 0)),
    ],
    out_specs=pl.BlockSpec((_BN, _C), lambda i: (i, 0)),
    out_shape=jax.ShapeDtypeStruct((_N, _C), jnp.float32),
)


def kernel(x, edge_index, edge_weight, W1, b1, W2, b2):
    src = edge_index[0].astype(jnp.int32)
    dst = edge_index[1].astype(jnp.int32)
    w = edge_weight.astype(jnp.float32)

    pad = _EP - _E
    fill = jnp.asarray((np.arange(pad, dtype=np.int32) * 13) % _N)
    srcp = jnp.concatenate([src, fill]).reshape(_EP // 128, 128)
    dstp = jnp.concatenate([dst, fill]).reshape(_EP // 128, 128)
    wp = jnp.concatenate([w, jnp.zeros((pad,), jnp.float32)])

    degp = _deg_kernel(dstp, wp)
    deg2 = jnp.stack([degp[:_N], degp[_NP:_NP + _N]], axis=1)

    hp = _tc_b(deg2, x, W1)

    a0, a1 = _agg_kernel(hp, srcp, dstp, wp)

    W2p = jnp.zeros((_H, _H), jnp.float32).at[:, :_C].set(W2)
    x1, gp = _tc_d(deg2, a0, a1, hp, b1.reshape(1, _H), W2p)

    c0, c1 = _agg_kernel(gp, srcp, dstp, wp)

    b2p = jnp.zeros((1, _H), jnp.float32).at[0, :_C].set(b2)
    out = _tc_f(deg2, c0, c1, gp, b2p)
    return (out, x1)


# speedup vs baseline: 1.6368x; 1.0274x over previous
import functools

import numpy as np

import jax
import jax.numpy as jnp
from jax import lax
from jax.experimental import pallas as pl
from jax.experimental.pallas import tpu as pltpu
from jax.experimental.pallas import tpu_sc as plsc

_N = 10000
_E = 320000
_D = 128
_H = 16
_C = 10

_NW = 32
_CH = 1024
_RB = _CH // 128
_PW = 10240
_EP = _NW * _PW
_NCH = _PW // _CH
_NP = 10240
_NPS = _NP // 16

_mesh = plsc.VectorSubcoreMesh(core_axis_name="c", subcore_axis_name="s")


@functools.partial(
    pl.kernel,
    out_type=jax.ShapeDtypeStruct((2 * _NP,), jnp.float32),
    scratch_types=[
        pltpu.VMEM((2, _RB, 128), jnp.int32),
        pltpu.VMEM((2, _CH), jnp.float32),
        pltpu.VMEM((_NPS,), jnp.float32),
        pltpu.VMEM_SHARED((_NP,), jnp.float32),
        pltpu.SemaphoreType.DMA((2,)),
        pltpu.SemaphoreType.DMA((2,)),
    ],
    mesh=_mesh,
    compiler_params=pltpu.CompilerParams(use_tc_tiling_on_sc=False),
)
def _deg_kernel(dst_hbm, w_hbm, out_hbm, dst_ref, w_ref, zb, deg_sh, gsem, ssem):
    c = lax.axis_index("c")
    s = lax.axis_index("s")
    wid = c * 16 + s

    def zrow(i, carry):
        zb[pl.ds(i * 16, 16)] = jnp.zeros((16,), jnp.float32)
        return carry

    lax.fori_loop(0, _NPS // 16, zrow, 0)
    pltpu.sync_copy(zb, deg_sh.at[pl.ds(s * _NPS, _NPS)])
    plsc.subcore_barrier()

    def stage(ci, b):
        e0 = wid * _PW + ci * _CH
        r0 = wid * (_PW // 128) + ci * _RB
        return [
            pltpu.async_copy(dst_hbm.at[pl.ds(r0, _RB)], dst_ref.at[b],
                             gsem.at[b]),
            pltpu.async_copy(w_hbm.at[pl.ds(e0, _CH)], w_ref.at[b],
                             gsem.at[b]),
        ]

    stage_cps = {0: stage(0, 0)}
    scatter_cps = {}
    for ci in range(_NCH):
        b = ci % 2
        for cp in stage_cps.pop(ci):
            cp.wait()
        if ci + 1 < _NCH:
            if ci - 1 >= 0:
                for cp in scatter_cps.pop(ci - 1):
                    cp.wait()
            stage_cps[ci + 1] = stage(ci + 1, 1 - b)
        scatter_cps[ci] = [
            pltpu.async_copy(
                w_ref.at[b, pl.ds(j * 128, 128)],
                deg_sh.at[dst_ref.at[b, j]],
                ssem.at[b],
                add=True,
            )
            for j in range(_RB)
        ]
    for ci in sorted(scatter_cps):
        for cp in scatter_cps[ci]:
            cp.wait()
    plsc.subcore_barrier()
    pltpu.sync_copy(
        deg_sh.at[pl.ds(s * _NPS, _NPS)],
        out_hbm.at[pl.ds(c * _NP + s * _NPS, _NPS)],
    )


_NRS = _N // 16


@functools.partial(
    pl.kernel,
    out_type=[
        jax.ShapeDtypeStruct((_N, _H), jnp.float32),
        jax.ShapeDtypeStruct((_N, _H), jnp.float32),
    ],
    scratch_types=[
        pltpu.VMEM((2, _RB, 128), jnp.int32),
        pltpu.VMEM((2, _RB, 128), jnp.int32),
        pltpu.VMEM((2, _CH), jnp.float32),
        pltpu.VMEM((2, _CH, _H), jnp.float32),
        pltpu.VMEM_SHARED((_N, _H), jnp.float32),
        pltpu.SemaphoreType.DMA((2,)),
        pltpu.SemaphoreType.DMA((2,)),
    ],
    mesh=_mesh,
    compiler_params=pltpu.CompilerParams(
        use_tc_tiling_on_sc=False, needs_layout_passes=False),
)
def _agg_kernel(tab_hbm, src_hbm, dst_hbm, w_hbm, out0_hbm, out1_hbm,
                src_ref, dst_ref, w_ref, rows_ref, acc_sh, gsem, ssem):
    c = lax.axis_index("c")
    s = lax.axis_index("s")
    wid = c * 16 + s

    def zrow(i, carry):
        rows_ref[0, i, :] = jnp.zeros((_H,), jnp.float32)
        return carry

    lax.fori_loop(0, _NRS, zrow, 0)
    pltpu.sync_copy(rows_ref.at[0, pl.ds(0, _NRS)],
                    acc_sh.at[pl.ds(s * _NRS, _NRS)])
    plsc.subcore_barrier()

    def stage_and_gather(ci, b):
        e0 = wid * _PW + ci * _CH
        r0 = wid * (_PW // 128) + ci * _RB
        pltpu.sync_copy(src_hbm.at[pl.ds(r0, _RB)], src_ref.at[b])
        pltpu.sync_copy(dst_hbm.at[pl.ds(r0, _RB)], dst_ref.at[b])
        pltpu.sync_copy(w_hbm.at[pl.ds(e0, _CH)], w_ref.at[b])
        return [
            pltpu.async_copy(
                tab_hbm.at[src_ref.at[b, j]],
                rows_ref.at[b, pl.ds(j * 128, 128)],
                gsem.at[b],
            )
            for j in range(_RB)
        ]

    gather_cps = {0: stage_and_gather(0, 0)}
    scatter_cps = {}
    for ci in range(_NCH):
        b = ci % 2
        for cp in gather_cps.pop(ci):
            cp.wait()
        if ci + 1 < _NCH:
            if ci - 1 >= 0:
                for cp in scatter_cps.pop(ci - 1):
                    cp.wait()
            gather_cps[ci + 1] = stage_and_gather(ci + 1, 1 - b)

        @plsc.parallel_loop(0, _CH, step=16, unroll=2)
        def _scale(e, b=b):
            wv = w_ref[b, pl.ds(e, 16)]
            for k in range(16):
                rows_ref[b, e + k, :] = rows_ref[b, e + k, :] * wv[k]
        scatter_cps[ci] = [
            pltpu.async_copy(
                rows_ref.at[b, pl.ds(j * 128, 128)],
                acc_sh.at[dst_ref.at[b, j]],
                ssem.at[b],
                add=True,
            )
            for j in range(_RB)
        ]
    for ci in sorted(scatter_cps):
        for cp in scatter_cps[ci]:
            cp.wait()
    plsc.subcore_barrier()

    @pl.when(c == 0)
    def _():
        pltpu.sync_copy(acc_sh.at[pl.ds(s * _NRS, _NRS)],
                        out0_hbm.at[pl.ds(s * _NRS, _NRS)])

    @pl.when(c == 1)
    def _():
        pltpu.sync_copy(acc_sh.at[pl.ds(s * _NRS, _NRS)],
                        out1_hbm.at[pl.ds(s * _NRS, _NRS)])


_BN = 5000


def _dinv_of(deg_ref):
    deg = deg_ref[:, 0:1] + deg_ref[:, 1:2] + 1.0
    return jnp.where(deg > 0, lax.rsqrt(jnp.maximum(deg, 1e-12)), 0.0)


def _tcb_body(deg_ref, x_ref, w1_ref, hp_ref):
    h = jnp.dot(x_ref[:, :], w1_ref[:, :], preferred_element_type=jnp.float32)
    hp_ref[:, :] = h * _dinv_of(deg_ref)


_tc_b = pl.pallas_call(
    _tcb_body,
    grid=(_N // _BN,),
    in_specs=[
        pl.BlockSpec((_BN, 2), lambda i: (i, 0)),
        pl.BlockSpec((_BN, _D), lambda i: (i, 0)),
        pl.BlockSpec((_D, _H), lambda i: (0, 0)),
    ],
    out_specs=pl.BlockSpec((_BN, _H), lambda i: (i, 0)),
    out_shape=jax.ShapeDtypeStruct((_N, _H), jnp.float32),
)


def _tcd_body(deg_ref, a0_ref, a1_ref, hp_ref, b1_ref, w2_ref, x1_ref, gp_ref):
    dinv = _dinv_of(deg_ref)
    x1 = dinv * (a0_ref[:, :] + a1_ref[:, :] + hp_ref[:, :]) + b1_ref[:, :]
    x1_ref[:, :] = x1
    r = jnp.maximum(x1, 0.0)
    g = jnp.dot(r, w2_ref[:, :], preferred_element_type=jnp.float32)
    gp_ref[:, :] = g * dinv


_tc_d = pl.pallas_call(
    _tcd_body,
    grid=(_N // _BN,),
    in_specs=[
        pl.BlockSpec((_BN, 2), lambda i: (i, 0)),
        pl.BlockSpec((_BN, _H), lambda i: (i, 0)),
        pl.BlockSpec((_BN, _H), lambda i: (i, 0)),
        pl.BlockSpec((_BN, _H), lambda i: (i, 0)),
        pl.BlockSpec((1, _H), lambda i: (0, 0)),
        pl.BlockSpec((_H, _H), lambda i: (0, 0)),
    ],
    out_specs=[
        pl.BlockSpec((_BN, _H), lambda i: (i, 0)),
        pl.BlockSpec((_BN, _H), lambda i: (i, 0)),
    ],
    out_shape=[
        jax.ShapeDtypeStruct((_N, _H), jnp.float32),
        jax.ShapeDtypeStruct((_N, _H), jnp.float32),
    ],
)


def _tcf_body(deg_ref, a0_ref, a1_ref, gp_ref, b2_ref, out_ref):
    dinv = _dinv_of(deg_ref)
    o = dinv * (a0_ref[:, :] + a1_ref[:, :] + gp_ref[:, :]) + b2_ref[:, :]
    mask = lax.broadcasted_iota(jnp.int32, (_BN, _H), 1) < _C
    z = jnp.where(mask, o, -3.0e38)
    m = jnp.max(z, axis=1, keepdims=True)
    e = jnp.where(mask, jnp.exp(z - m), 0.0)
    lse = jnp.log(jnp.sum(e, axis=1, keepdims=True)) + m
    out_ref[:, :] = lax.slice(o - lse, (0, 0), (_BN, _C))


_tc_f = pl.pallas_call(
    _tcf_body,
    grid=(_N // _BN,),
    in_specs=[
        pl.BlockSpec((_BN, 2), lambda i: (i, 0)),
        pl.BlockSpec((_BN, _H), lambda i: (i, 0)),
        pl.BlockSpec((_BN, _H), lambda i: (i, 0)),
        pl.BlockSpec((_BN, _H), lambda i: (i, 0)),
        pl.BlockSpec((1, _H), lambda i: (0, 0)),
    ],
    out_specs=pl.BlockSpec((_BN, _C), lambda i: (i, 0)),
    out_shape=jax.ShapeDtypeStruct((_N, _C), jnp.float32),
)


def kernel(x, edge_index, edge_weight, W1, b1, W2, b2):
    src = edge_index[0].astype(jnp.int32)
    dst = edge_index[1].astype(jnp.int32)
    w = edge_weight.astype(jnp.float32)

    pad = _EP - _E
    fill = jnp.asarray((np.arange(pad, dtype=np.int32) * 13) % _N)
    srcp = jnp.concatenate([src, fill]).reshape(_EP // 128, 128)
    dstp = jnp.concatenate([dst, fill]).reshape(_EP // 128, 128)
    wp = jnp.concatenate([w, jnp.zeros((pad,), jnp.float32)])

    degp = _deg_kernel(dstp, wp)
    deg2 = jnp.stack([degp[:_N], degp[_NP:_NP + _N]], axis=1)

    hp = _tc_b(deg2, x, W1)

    a0, a1 = _agg_kernel(hp, srcp, dstp, wp)

    W2p = jnp.zeros((_H, _H), jnp.float32).at[:, :_C].set(W2)
    x1, gp = _tc_d(deg2, a0, a1, hp, b1.reshape(1, _H), W2p)

    c0, c1 = _agg_kernel(gp, srcp, dstp, wp)

    b2p = jnp.zeros((1, _H), jnp.float32).at[0, :_C].set(b2)
    out = _tc_f(deg2, c0, c1, gp, b2p)
    return (out, x1)


# speedup vs baseline: 2.0777x; 1.2693x over previous
import functools

import numpy as np

import jax
import jax.numpy as jnp
from jax import lax
from jax.experimental import pallas as pl
from jax.experimental.pallas import tpu as pltpu
from jax.experimental.pallas import tpu_sc as plsc

_N = 10000
_E = 320000
_D = 128
_H = 16
_C = 10

_NW = 32
_CH = 1024
_RB = _CH // 128
_EW = _E // _NW
_NP = 10240
_NPS = _NP // 16

_CHUNKS = []
_off = 0
while _off < _EW:
    _ch = min(_CH, _EW - _off)
    _bs, _r = [], _ch
    while _r > 0:
        _bs.append(min(128, _r))
        _r -= _bs[-1]
    _CHUNKS.append((_off, _ch, _bs))
    _off += _ch
_NCH = len(_CHUNKS)

_mesh = plsc.VectorSubcoreMesh(core_axis_name="c", subcore_axis_name="s")


@functools.partial(
    pl.kernel,
    out_type=jax.ShapeDtypeStruct((2 * _NP,), jnp.float32),
    scratch_types=[
        pltpu.VMEM((2, _RB, 128), jnp.int32),
        pltpu.VMEM((2, 16), jnp.int32),
        pltpu.VMEM((2, _CH), jnp.float32),
        pltpu.VMEM((_NPS,), jnp.float32),
        pltpu.VMEM_SHARED((_NP,), jnp.float32),
        pltpu.SemaphoreType.DMA((2,)),
        pltpu.SemaphoreType.DMA((2,)),
    ],
    mesh=_mesh,
    compiler_params=pltpu.CompilerParams(use_tc_tiling_on_sc=False),
)
def _deg_kernel(ei_hbm, w_hbm, out_hbm, dst_ref, dstt_ref, w_ref, zb, deg_sh,
                gsem, ssem):
    c = lax.axis_index("c")
    s = lax.axis_index("s")
    wid = c * 16 + s

    def zrow(i, carry):
        zb[pl.ds(i * 16, 16)] = jnp.zeros((16,), jnp.float32)
        return carry

    lax.fori_loop(0, _NPS // 16, zrow, 0)
    pltpu.sync_copy(zb, deg_sh.at[pl.ds(s * _NPS, _NPS)])
    plsc.subcore_barrier()

    def stage(ci, b):
        off, ch, bs = _CHUNKS[ci]
        e0 = wid * _EW + off
        cps = [pltpu.async_copy(w_hbm.at[pl.ds(e0, ch)],
                                w_ref.at[b, pl.ds(0, ch)], gsem.at[b])]
        pos = 0
        for j, t in enumerate(bs):
            tgt = dst_ref.at[b, j] if t == 128 else dstt_ref.at[b]
            cps.append(pltpu.async_copy(ei_hbm.at[1, pl.ds(e0 + pos, t)],
                                        tgt, gsem.at[b]))
            pos += t
        return cps

    stage_cps = {0: stage(0, 0)}
    scatter_cps = {}
    for ci in range(_NCH):
        b = ci % 2
        for cp in stage_cps.pop(ci):
            cp.wait()
        if ci + 1 < _NCH:
            if ci - 1 >= 0:
                for cp in scatter_cps.pop(ci - 1):
                    cp.wait()
            stage_cps[ci + 1] = stage(ci + 1, 1 - b)
        _, _, bs = _CHUNKS[ci]
        cps = []
        pos = 0
        for j, t in enumerate(bs):
            idx = dst_ref.at[b, j] if t == 128 else dstt_ref.at[b]
            cps.append(pltpu.async_copy(
                w_ref.at[b, pl.ds(pos, t)],
                deg_sh.at[idx],
                ssem.at[b],
                add=True,
            ))
            pos += t
        scatter_cps[ci] = cps
    for ci in sorted(scatter_cps):
        for cp in scatter_cps[ci]:
            cp.wait()
    plsc.subcore_barrier()
    pltpu.sync_copy(
        deg_sh.at[pl.ds(s * _NPS, _NPS)],
        out_hbm.at[pl.ds(c * _NP + s * _NPS, _NPS)],
    )


_NRS = _N // 16


@functools.partial(
    pl.kernel,
    out_type=[
        jax.ShapeDtypeStruct((_N, _H), jnp.float32),
        jax.ShapeDtypeStruct((_N, _H), jnp.float32),
    ],
    scratch_types=[
        pltpu.VMEM((3, _CH), jnp.int32),
        pltpu.VMEM((3, _RB, 128), jnp.int32),
        pltpu.VMEM((3, 16), jnp.int32),
        pltpu.VMEM((3, _CH), jnp.float32),
        pltpu.VMEM((2, _CH, _H), jnp.float32),
        pltpu.VMEM_SHARED((_N, _H), jnp.float32),
        pltpu.SemaphoreType.DMA((3,)),
        pltpu.SemaphoreType.DMA((2,)),
        pltpu.SemaphoreType.DMA((2,)),
    ],
    mesh=_mesh,
    compiler_params=pltpu.CompilerParams(
        use_tc_tiling_on_sc=False, needs_layout_passes=False),
)
def _agg_kernel(tab_hbm, ei_hbm, w_hbm, out0_hbm, out1_hbm,
                src_ref, dst_ref, dstt_ref, w_ref, rows_ref, acc_sh,
                tsem, gsem, ssem):
    c = lax.axis_index("c")
    s = lax.axis_index("s")
    wid = c * 16 + s

    def zrow(i, carry):
        rows_ref[0, i, :] = jnp.zeros((_H,), jnp.float32)
        return carry

    lax.fori_loop(0, _NRS, zrow, 0)
    pltpu.sync_copy(rows_ref.at[0, pl.ds(0, _NRS)],
                    acc_sh.at[pl.ds(s * _NRS, _NRS)])
    plsc.subcore_barrier()

    def stage(ci, t3):
        off, ch, bs = _CHUNKS[ci]
        e0 = wid * _EW + off
        cps = [
            pltpu.async_copy(ei_hbm.at[0, pl.ds(e0, ch)],
                             src_ref.at[t3, pl.ds(0, ch)], tsem.at[t3]),
            pltpu.async_copy(w_hbm.at[pl.ds(e0, ch)],
                             w_ref.at[t3, pl.ds(0, ch)], tsem.at[t3]),
        ]
        pos = 0
        for j, t in enumerate(bs):
            tgt = dst_ref.at[t3, j] if t == 128 else dstt_ref.at[t3]
            cps.append(pltpu.async_copy(ei_hbm.at[1, pl.ds(e0 + pos, t)],
                                        tgt, tsem.at[t3]))
            pos += t
        return cps

    def gather(ci, t3, rb):
        _, _, bs = _CHUNKS[ci]
        cps = []
        pos = 0
        for t in bs:
            cps.append(pltpu.async_copy(
                tab_hbm.at[src_ref.at[t3, pl.ds(pos, t)]],
                rows_ref.at[rb, pl.ds(pos, t)],
                gsem.at[rb],
            ))
            pos += t
        return cps

    stage_cps = {0: stage(0, 0), 1: stage(1, 1)}
    for cp in stage_cps.pop(0):
        cp.wait()
    gather_cps = {0: gather(0, 0, 0)}
    scatter_cps = {}
    for ci in range(_NCH):
        rb = ci % 2
        t3 = ci % 3
        for cp in gather_cps.pop(ci):
            cp.wait()
        if ci - 1 >= 0:
            for cp in scatter_cps.pop(ci - 1):
                cp.wait()
        if ci + 2 < _NCH:
            stage_cps[ci + 2] = stage(ci + 2, (ci + 2) % 3)
        if ci + 1 < _NCH:
            for cp in stage_cps.pop(ci + 1):
                cp.wait()
            gather_cps[ci + 1] = gather(ci + 1, (ci + 1) % 3, 1 - rb)

        _, ch, bs = _CHUNKS[ci]

        @plsc.parallel_loop(0, ch, step=16, unroll=2 if ch % 32 == 0 else 1)
        def _scale(e, rb=rb, t3=t3):
            wv = w_ref[t3, pl.ds(e, 16)]
            for k in range(16):
                rows_ref[rb, e + k, :] = rows_ref[rb, e + k, :] * wv[k]

        cps = []
        pos = 0
        for j, t in enumerate(bs):
            idx = dst_ref.at[t3, j] if t == 128 else dstt_ref.at[t3]
            cps.append(pltpu.async_copy(
                rows_ref.at[rb, pl.ds(pos, t)],
                acc_sh.at[idx],
                ssem.at[rb],
                add=True,
            ))
            pos += t
        scatter_cps[ci] = cps
    for ci in sorted(scatter_cps):
        for cp in scatter_cps[ci]:
            cp.wait()
    plsc.subcore_barrier()

    @pl.when(c == 0)
    def _():
        pltpu.sync_copy(acc_sh.at[pl.ds(s * _NRS, _NRS)],
                        out0_hbm.at[pl.ds(s * _NRS, _NRS)])

    @pl.when(c == 1)
    def _():
        pltpu.sync_copy(acc_sh.at[pl.ds(s * _NRS, _NRS)],
                        out1_hbm.at[pl.ds(s * _NRS, _NRS)])


_BN = 5000


def _dinv_of(deg_ref):
    deg = deg_ref[:, 0:1] + deg_ref[:, 1:2] + 1.0
    return jnp.where(deg > 0, lax.rsqrt(jnp.maximum(deg, 1e-12)), 0.0)


def _tcb_body(deg_ref, x_ref, w1_ref, hp_ref):
    h = jnp.dot(x_ref[:, :], w1_ref[:, :], preferred_element_type=jnp.float32)
    hp_ref[:, :] = h * _dinv_of(deg_ref)


_tc_b = pl.pallas_call(
    _tcb_body,
    grid=(_N // _BN,),
    in_specs=[
        pl.BlockSpec((_BN, 2), lambda i: (i, 0)),
        pl.BlockSpec((_BN, _D), lambda i: (i, 0)),
        pl.BlockSpec((_D, _H), lambda i: (0, 0)),
    ],
    out_specs=pl.BlockSpec((_BN, _H), lambda i: (i, 0)),
    out_shape=jax.ShapeDtypeStruct((_N, _H), jnp.float32),
)


def _tcd_body(deg_ref, a0_ref, a1_ref, hp_ref, b1_ref, w2_ref, x1_ref, gp_ref):
    dinv = _dinv_of(deg_ref)
    x1 = dinv * (a0_ref[:, :] + a1_ref[:, :] + hp_ref[:, :]) + b1_ref[:, :]
    x1_ref[:, :] = x1
    r = jnp.maximum(x1, 0.0)
    g = jnp.dot(r, w2_ref[:, :], preferred_element_type=jnp.float32)
    gp_ref[:, :] = g * dinv


_tc_d = pl.pallas_call(
    _tcd_body,
    grid=(_N // _BN,),
    in_specs=[
        pl.BlockSpec((_BN, 2), lambda i: (i, 0)),
        pl.BlockSpec((_BN, _H), lambda i: (i, 0)),
        pl.BlockSpec((_BN, _H), lambda i: (i, 0)),
        pl.BlockSpec((_BN, _H), lambda i: (i, 0)),
        pl.BlockSpec((1, _H), lambda i: (0, 0)),
        pl.BlockSpec((_H, _H), lambda i: (0, 0)),
    ],
    out_specs=[
        pl.BlockSpec((_BN, _H), lambda i: (i, 0)),
        pl.BlockSpec((_BN, _H), lambda i: (i, 0)),
    ],
    out_shape=[
        jax.ShapeDtypeStruct((_N, _H), jnp.float32),
        jax.ShapeDtypeStruct((_N, _H), jnp.float32),
    ],
)


def _tcf_body(deg_ref, a0_ref, a1_ref, gp_ref, b2_ref, out_ref):
    dinv = _dinv_of(deg_ref)
    o = dinv * (a0_ref[:, :] + a1_ref[:, :] + gp_ref[:, :]) + b2_ref[:, :]
    mask = lax.broadcasted_iota(jnp.int32, (_BN, _H), 1) < _C
    z = jnp.where(mask, o, -3.0e38)
    m = jnp.max(z, axis=1, keepdims=True)
    e = jnp.where(mask, jnp.exp(z - m), 0.0)
    lse = jnp.log(jnp.sum(e, axis=1, keepdims=True)) + m
    out_ref[:, :] = lax.slice(o - lse, (0, 0), (_BN, _C))


_tc_f = pl.pallas_call(
    _tcf_body,
    grid=(_N // _BN,),
    in_specs=[
        pl.BlockSpec((_BN, 2), lambda i: (i, 0)),
        pl.BlockSpec((_BN, _H), lambda i: (i, 0)),
        pl.BlockSpec((_BN, _H), lambda i: (i, 0)),
        pl.BlockSpec((_BN, _H), lambda i: (i, 0)),
        pl.BlockSpec((1, _H), lambda i: (0, 0)),
    ],
    out_specs=pl.BlockSpec((_BN, _C), lambda i: (i, 0)),
    out_shape=jax.ShapeDtypeStruct((_N, _C), jnp.float32),
)


def kernel(x, edge_index, edge_weight, W1, b1, W2, b2):
    ei = edge_index.astype(jnp.int32)
    w = edge_weight.astype(jnp.float32)

    degp = _deg_kernel(ei, w)
    deg2 = jnp.stack([degp[:_N], degp[_NP:_NP + _N]], axis=1)

    hp = _tc_b(deg2, x, W1)

    a0, a1 = _agg_kernel(hp, ei, w)

    W2p = jnp.zeros((_H, _H), jnp.float32).at[:, :_C].set(W2)
    x1, gp = _tc_d(deg2, a0, a1, hp, b1.reshape(1, _H), W2p)

    c0, c1 = _agg_kernel(gp, ei, w)

    b2p = jnp.zeros((1, _H), jnp.float32).at[0, :_C].set(b2)
    out = _tc_f(deg2, c0, c1, gp, b2p)
    return (out, x1)


# speedup vs baseline: 2.1002x; 1.0108x over previous
import functools

import numpy as np

import jax
import jax.numpy as jnp
from jax import lax
from jax.experimental import pallas as pl
from jax.experimental.pallas import tpu as pltpu
from jax.experimental.pallas import tpu_sc as plsc

_N = 10000
_E = 320000
_D = 128
_H = 16
_C = 10

_NW = 32
_CH = 2048
_RB = _CH // 128
_EW = _E // _NW
_NP = 10240
_NPS = _NP // 16

_CHUNKS = []
_off = 0
while _off < _EW:
    _ch = min(_CH, _EW - _off)
    _bs, _r = [], _ch
    while _r > 0:
        _bs.append(min(128, _r))
        _r -= _bs[-1]
    _CHUNKS.append((_off, _ch, _bs))
    _off += _ch
_NCH = len(_CHUNKS)

_mesh = plsc.VectorSubcoreMesh(core_axis_name="c", subcore_axis_name="s")


@functools.partial(
    pl.kernel,
    out_type=jax.ShapeDtypeStruct((2 * _NP,), jnp.float32),
    scratch_types=[
        pltpu.VMEM((2, _RB, 128), jnp.int32),
        pltpu.VMEM((2, 16), jnp.int32),
        pltpu.VMEM((2, _CH), jnp.float32),
        pltpu.VMEM((_NPS,), jnp.float32),
        pltpu.VMEM_SHARED((_NP,), jnp.float32),
        pltpu.SemaphoreType.DMA((2,)),
        pltpu.SemaphoreType.DMA((2,)),
    ],
    mesh=_mesh,
    compiler_params=pltpu.CompilerParams(use_tc_tiling_on_sc=False),
)
def _deg_kernel(ei_hbm, w_hbm, out_hbm, dst_ref, dstt_ref, w_ref, zb, deg_sh,
                gsem, ssem):
    c = lax.axis_index("c")
    s = lax.axis_index("s")
    wid = c * 16 + s

    def zrow(i, carry):
        zb[pl.ds(i * 16, 16)] = jnp.zeros((16,), jnp.float32)
        return carry

    lax.fori_loop(0, _NPS // 16, zrow, 0)
    pltpu.sync_copy(zb, deg_sh.at[pl.ds(s * _NPS, _NPS)])
    plsc.subcore_barrier()

    def stage(ci, b):
        off, ch, bs = _CHUNKS[ci]
        e0 = wid * _EW + off
        cps = [pltpu.async_copy(w_hbm.at[pl.ds(e0, ch)],
                                w_ref.at[b, pl.ds(0, ch)], gsem.at[b])]
        pos = 0
        for j, t in enumerate(bs):
            tgt = dst_ref.at[b, j] if t == 128 else dstt_ref.at[b]
            cps.append(pltpu.async_copy(ei_hbm.at[1, pl.ds(e0 + pos, t)],
                                        tgt, gsem.at[b]))
            pos += t
        return cps

    stage_cps = {0: stage(0, 0)}
    scatter_cps = {}
    for ci in range(_NCH):
        b = ci % 2
        for cp in stage_cps.pop(ci):
            cp.wait()
        if ci + 1 < _NCH:
            if ci - 1 >= 0:
                for cp in scatter_cps.pop(ci - 1):
                    cp.wait()
            stage_cps[ci + 1] = stage(ci + 1, 1 - b)
        _, _, bs = _CHUNKS[ci]
        cps = []
        pos = 0
        for j, t in enumerate(bs):
            idx = dst_ref.at[b, j] if t == 128 else dstt_ref.at[b]
            cps.append(pltpu.async_copy(
                w_ref.at[b, pl.ds(pos, t)],
                deg_sh.at[idx],
                ssem.at[b],
                add=True,
            ))
            pos += t
        scatter_cps[ci] = cps
    for ci in sorted(scatter_cps):
        for cp in scatter_cps[ci]:
            cp.wait()
    plsc.subcore_barrier()
    pltpu.sync_copy(
        deg_sh.at[pl.ds(s * _NPS, _NPS)],
        out_hbm.at[pl.ds(c * _NP + s * _NPS, _NPS)],
    )


_NRS = _N // 16


@functools.partial(
    pl.kernel,
    out_type=[
        jax.ShapeDtypeStruct((_N, _H), jnp.float32),
        jax.ShapeDtypeStruct((_N, _H), jnp.float32),
    ],
    scratch_types=[
        pltpu.VMEM((3, _CH), jnp.int32),
        pltpu.VMEM((3, _RB, 128), jnp.int32),
        pltpu.VMEM((3, 16), jnp.int32),
        pltpu.VMEM((3, _CH), jnp.float32),
        pltpu.VMEM((2, _CH, _H), jnp.float32),
        pltpu.VMEM_SHARED((_N, _H), jnp.float32),
        pltpu.SemaphoreType.DMA((3,)),
        pltpu.SemaphoreType.DMA((2,)),
        pltpu.SemaphoreType.DMA((2,)),
    ],
    mesh=_mesh,
    compiler_params=pltpu.CompilerParams(
        use_tc_tiling_on_sc=False, needs_layout_passes=False),
)
def _agg_kernel(tab_hbm, ei_hbm, w_hbm, out0_hbm, out1_hbm,
                src_ref, dst_ref, dstt_ref, w_ref, rows_ref, acc_sh,
                tsem, gsem, ssem):
    c = lax.axis_index("c")
    s = lax.axis_index("s")
    wid = c * 16 + s

    def zrow(i, carry):
        rows_ref[0, i, :] = jnp.zeros((_H,), jnp.float32)
        return carry

    lax.fori_loop(0, _NRS, zrow, 0)
    pltpu.sync_copy(rows_ref.at[0, pl.ds(0, _NRS)],
                    acc_sh.at[pl.ds(s * _NRS, _NRS)])
    plsc.subcore_barrier()

    def stage(ci, t3):
        off, ch, bs = _CHUNKS[ci]
        e0 = wid * _EW + off
        cps = [
            pltpu.async_copy(ei_hbm.at[0, pl.ds(e0, ch)],
                             src_ref.at[t3, pl.ds(0, ch)], tsem.at[t3]),
            pltpu.async_copy(w_hbm.at[pl.ds(e0, ch)],
                             w_ref.at[t3, pl.ds(0, ch)], tsem.at[t3]),
        ]
        pos = 0
        for j, t in enumerate(bs):
            tgt = dst_ref.at[t3, j] if t == 128 else dstt_ref.at[t3]
            cps.append(pltpu.async_copy(ei_hbm.at[1, pl.ds(e0 + pos, t)],
                                        tgt, tsem.at[t3]))
            pos += t
        return cps

    def gather(ci, t3, rb):
        _, _, bs = _CHUNKS[ci]
        cps = []
        pos = 0
        for t in bs:
            cps.append(pltpu.async_copy(
                tab_hbm.at[src_ref.at[t3, pl.ds(pos, t)]],
                rows_ref.at[rb, pl.ds(pos, t)],
                gsem.at[rb],
            ))
            pos += t
        return cps

    stage_cps = {0: stage(0, 0), 1: stage(1, 1)}
    for cp in stage_cps.pop(0):
        cp.wait()
    gather_cps = {0: gather(0, 0, 0)}
    scatter_cps = {}
    for ci in range(_NCH):
        rb = ci % 2
        t3 = ci % 3
        for cp in gather_cps.pop(ci):
            cp.wait()
        if ci - 1 >= 0:
            for cp in scatter_cps.pop(ci - 1):
                cp.wait()
        if ci + 2 < _NCH:
            stage_cps[ci + 2] = stage(ci + 2, (ci + 2) % 3)
        if ci + 1 < _NCH:
            for cp in stage_cps.pop(ci + 1):
                cp.wait()
            gather_cps[ci + 1] = gather(ci + 1, (ci + 1) % 3, 1 - rb)

        _, ch, bs = _CHUNKS[ci]

        @plsc.parallel_loop(0, ch, step=16, unroll=4 if ch % 64 == 0 else 1)
        def _scale(e, rb=rb, t3=t3):
            wv = w_ref[t3, pl.ds(e, 16)]
            for k in range(16):
                rows_ref[rb, e + k, :] = rows_ref[rb, e + k, :] * wv[k]

        cps = []
        pos = 0
        for j, t in enumerate(bs):
            idx = dst_ref.at[t3, j] if t == 128 else dstt_ref.at[t3]
            cps.append(pltpu.async_copy(
                rows_ref.at[rb, pl.ds(pos, t)],
                acc_sh.at[idx],
                ssem.at[rb],
                add=True,
            ))
            pos += t
        scatter_cps[ci] = cps
    for ci in sorted(scatter_cps):
        for cp in scatter_cps[ci]:
            cp.wait()
    plsc.subcore_barrier()

    @pl.when(c == 0)
    def _():
        pltpu.sync_copy(acc_sh.at[pl.ds(s * _NRS, _NRS)],
                        out0_hbm.at[pl.ds(s * _NRS, _NRS)])

    @pl.when(c == 1)
    def _():
        pltpu.sync_copy(acc_sh.at[pl.ds(s * _NRS, _NRS)],
                        out1_hbm.at[pl.ds(s * _NRS, _NRS)])


_BN = 5000


def _dinv_of(deg_ref):
    deg = deg_ref[:, 0:1] + deg_ref[:, 1:2] + 1.0
    return jnp.where(deg > 0, lax.rsqrt(jnp.maximum(deg, 1e-12)), 0.0)


def _tcb_body(deg_ref, x_ref, w1_ref, hp_ref):
    h = jnp.dot(x_ref[:, :], w1_ref[:, :], preferred_element_type=jnp.float32)
    hp_ref[:, :] = h * _dinv_of(deg_ref)


_tc_b = pl.pallas_call(
    _tcb_body,
    grid=(_N // _BN,),
    in_specs=[
        pl.BlockSpec((_BN, 2), lambda i: (i, 0)),
        pl.BlockSpec((_BN, _D), lambda i: (i, 0)),
        pl.BlockSpec((_D, _H), lambda i: (0, 0)),
    ],
    out_specs=pl.BlockSpec((_BN, _H), lambda i: (i, 0)),
    out_shape=jax.ShapeDtypeStruct((_N, _H), jnp.float32),
)


def _tcd_body(deg_ref, a0_ref, a1_ref, hp_ref, b1_ref, w2_ref, x1_ref, gp_ref):
    dinv = _dinv_of(deg_ref)
    x1 = dinv * (a0_ref[:, :] + a1_ref[:, :] + hp_ref[:, :]) + b1_ref[:, :]
    x1_ref[:, :] = x1
    r = jnp.maximum(x1, 0.0)
    g = jnp.dot(r, w2_ref[:, :], preferred_element_type=jnp.float32)
    gp_ref[:, :] = g * dinv


_tc_d = pl.pallas_call(
    _tcd_body,
    grid=(_N // _BN,),
    in_specs=[
        pl.BlockSpec((_BN, 2), lambda i: (i, 0)),
        pl.BlockSpec((_BN, _H), lambda i: (i, 0)),
        pl.BlockSpec((_BN, _H), lambda i: (i, 0)),
        pl.BlockSpec((_BN, _H), lambda i: (i, 0)),
        pl.BlockSpec((1, _H), lambda i: (0, 0)),
        pl.BlockSpec((_H, _H), lambda i: (0, 0)),
    ],
    out_specs=[
        pl.BlockSpec((_BN, _H), lambda i: (i, 0)),
        pl.BlockSpec((_BN, _H), lambda i: (i, 0)),
    ],
    out_shape=[
        jax.ShapeDtypeStruct((_N, _H), jnp.float32),
        jax.ShapeDtypeStruct((_N, _H), jnp.float32),
    ],
)


def _tcf_body(deg_ref, a0_ref, a1_ref, gp_ref, b2_ref, out_ref):
    dinv = _dinv_of(deg_ref)
    o = dinv * (a0_ref[:, :] + a1_ref[:, :] + gp_ref[:, :]) + b2_ref[:, :]
    mask = lax.broadcasted_iota(jnp.int32, (_BN, _H), 1) < _C
    z = jnp.where(mask, o, -3.0e38)
    m = jnp.max(z, axis=1, keepdims=True)
    e = jnp.where(mask, jnp.exp(z - m), 0.0)
    lse = jnp.log(jnp.sum(e, axis=1, keepdims=True)) + m
    out_ref[:, :] = lax.slice(o - lse, (0, 0), (_BN, _C))


_tc_f = pl.pallas_call(
    _tcf_body,
    grid=(_N // _BN,),
    in_specs=[
        pl.BlockSpec((_BN, 2), lambda i: (i, 0)),
        pl.BlockSpec((_BN, _H), lambda i: (i, 0)),
        pl.BlockSpec((_BN, _H), lambda i: (i, 0)),
        pl.BlockSpec((_BN, _H), lambda i: (i, 0)),
        pl.BlockSpec((1, _H), lambda i: (0, 0)),
    ],
    out_specs=pl.BlockSpec((_BN, _C), lambda i: (i, 0)),
    out_shape=jax.ShapeDtypeStruct((_N, _C), jnp.float32),
)


def kernel(x, edge_index, edge_weight, W1, b1, W2, b2):
    ei = edge_index.astype(jnp.int32)
    w = edge_weight.astype(jnp.float32)

    degp = _deg_kernel(ei, w)
    deg2 = jnp.stack([degp[:_N], degp[_NP:_NP + _N]], axis=1)

    hp = _tc_b(deg2, x, W1)

    a0, a1 = _agg_kernel(hp, ei, w)

    W2p = jnp.zeros((_H, _H), jnp.float32).at[:, :_C].set(W2)
    x1, gp = _tc_d(deg2, a0, a1, hp, b1.reshape(1, _H), W2p)

    c0, c1 = _agg_kernel(gp, ei, w)

    b2p = jnp.zeros((1, _H), jnp.float32).at[0, :_C].set(b2)
    out = _tc_f(deg2, c0, c1, gp, b2p)
    return (out, x1)
